# speedup vs baseline: 117.3954x; 117.3954x over previous
import functools

import jax
import jax.numpy as jnp
from jax import lax
from jax.experimental import pallas as pl
from jax.experimental.pallas import tpu as pltpu
from jax.experimental.pallas import tpu_sc as plsc

N = 4096
E = 16384
D = 128
NR = 4
KWIN = 64
NWORK = 32
EW = E // NWORK
BE = 256
NBLK = E // BE

_f32 = jnp.float32
_i32 = jnp.int32


def _sc_gather(big, heads, tails):
    mesh = plsc.VectorSubcoreMesh(core_axis_name="c", subcore_axis_name="s")

    @functools.partial(
        pl.kernel,
        out_type=jax.ShapeDtypeStruct((14 * E, 128), _f32),
        mesh=mesh,
        scratch_types=[
            pltpu.VMEM((EW,), _i32),
            pltpu.VMEM((EW,), _i32),
            pltpu.VMEM((4, 128), _i32),
            pltpu.VMEM((EW, 128), _f32),
            pltpu.SemaphoreType.DMA,
        ],
    )
    def k(big_hbm, heads_hbm, tails_hbm, out_hbm, hv, tv, idx, rows, sem):
        wid = lax.axis_index("s") * 2 + lax.axis_index("c")
        base_e = wid * EW
        pltpu.sync_copy(heads_hbm.at[pl.ds(base_e, EW)], hv)
        pltpu.sync_copy(tails_hbm.at[pl.ds(base_e, EW)], tv)
        for combo in range(14):
            f = combo % 7
            src_v = hv if combo < 7 else tv
            for j in range(EW // 16):
                ch, off = divmod(j, 8)
                idx[ch, pl.ds(off * 16, 16)] = (
                    src_v[pl.ds(j * 16, 16)] + f * N)
            cps = [
                pltpu.async_copy(big_hbm.at[idx.at[ch]],
                                 rows.at[pl.ds(ch * 128, 128)], sem)
                for ch in range(4)
            ]
            for cp in cps:
                cp.wait()
            pltpu.sync_copy(rows, out_hbm.at[pl.ds(combo * E + base_e, EW)])

    return k(big, heads, tails)


def _sc_scatter(scores, rank_s, seg_s, rank_d, seg_d, lo_vec):
    mesh = plsc.VectorSubcoreMesh(core_axis_name="c", subcore_axis_name="s")
    sds = jax.ShapeDtypeStruct((KWIN * N + NWORK, 128), _f32)

    @functools.partial(
        pl.kernel,
        out_type=(sds, sds),
        mesh=mesh,
        scratch_types=[
            pltpu.VMEM((EW,), _i32),
            pltpu.VMEM((EW,), _i32),
            pltpu.VMEM((EW,), _i32),
            pltpu.VMEM((EW,), _i32),
            pltpu.VMEM((16,), _i32),
            pltpu.VMEM((4, 128), _i32),
            pltpu.VMEM((4, 128), _i32),
            pltpu.VMEM((EW, 128), _f32),
            pltpu.SemaphoreType.DMA,
        ],
    )
    def k(sc_hbm, rs_hbm, ss_hbm, rd_hbm, sd_hbm, lo_hbm, out_s, out_d,
          rs, ss, rd, sd_, lov, idx_s, idx_d, rows, sem):
        wid = lax.axis_index("s") * 2 + lax.axis_index("c")
        base_e = wid * EW
        pltpu.sync_copy(sc_hbm.at[pl.ds(base_e, EW)], rows)
        pltpu.sync_copy(rs_hbm.at[pl.ds(base_e, EW)], rs)
        pltpu.sync_copy(ss_hbm.at[pl.ds(base_e, EW)], ss)
        pltpu.sync_copy(rd_hbm.at[pl.ds(base_e, EW)], rd)
        pltpu.sync_copy(sd_hbm.at[pl.ds(base_e, EW)], sd_)
        pltpu.sync_copy(lo_hbm, lov)
        lo = lov[pl.ds(0, 16)]
        for j in range(EW // 16):
            ch, off = divmod(j, 8)
            rel = rs[pl.ds(j * 16, 16)] - lo
            ok = (rel >= 0) & (rel < KWIN)
            idx_s[ch, pl.ds(off * 16, 16)] = jnp.where(
                ok, rel * N + ss[pl.ds(j * 16, 16)], KWIN * N + wid)
            rel = rd[pl.ds(j * 16, 16)] - lo
            ok = (rel >= 0) & (rel < KWIN)
            idx_d[ch, pl.ds(off * 16, 16)] = jnp.where(
                ok, rel * N + sd_[pl.ds(j * 16, 16)], KWIN * N + wid)
        cps = []
        for ch in range(4):
            cps.append(pltpu.async_copy(
                rows.at[pl.ds(ch * 128, 128)], out_s.at[idx_s.at[ch]], sem))
            cps.append(pltpu.async_copy(
                rows.at[pl.ds(ch * 128, 128)], out_d.at[idx_d.at[ch]], sem))
        for cp in cps:
            cp.wait()

    return k(scores, rank_s, seg_s, rank_d, seg_d, lo_vec)


def _scores_kernel(g_ref, wk_ref, dy_ref, rl_ref, re_ref, out_ref):
    wk = wk_ref[...]
    dy = dy_ref[...]
    g = g_ref[...]
    th = (g[3, 0] * jnp.sin(g[1, 0] * wk + g[2, 0])
          + g[6, 0] * jnp.sin(g[4, 0] * dy + g[5, 0]))
    tt = (g[10, 0] * jnp.sin(g[8, 0] * wk + g[9, 0])
          + g[13, 0] * jnp.sin(g[11, 0] * dy + g[12, 0]))
    rion = lax.broadcasted_iota(_i32, (BE, NR), 1)
    oh = (rl_ref[...] == rion).astype(_f32)
    r = jnp.dot(oh, re_ref[...], preferred_element_type=_f32)
    ent = (g[0, 0] * g[7, 0] * r)[:, :46]
    tim = (th * tt)[:, :82] * r[:, 46:128]
    out_ref[...] = jnp.concatenate([ent, tim], axis=1)


def _tc_scores(gath, weeks2, days2, rels2, rel_embs):
    gr = gath.reshape(14, NBLK, BE, 128)
    return pl.pallas_call(
        _scores_kernel,
        grid=(NBLK,),
        in_specs=[
            pl.BlockSpec((14, 1, BE, 128), lambda i: (0, i, 0, 0)),
            pl.BlockSpec((BE, 1), lambda i: (i, 0)),
            pl.BlockSpec((BE, 1), lambda i: (i, 0)),
            pl.BlockSpec((BE, 1), lambda i: (i, 0)),
            pl.BlockSpec((NR, 128), lambda i: (0, 0)),
        ],
        out_specs=pl.BlockSpec((BE, 128), lambda i: (i, 0)),
        out_shape=jax.ShapeDtypeStruct((E, 128), _f32),
    )(gr, weeks2, days2, rels2, rel_embs)


def _ranks_kernel(sc_ref, sr_ref, dc_ref, dr_ref,
                  rnks_ref, rnkd_ref, cnts_ref, cntd_ref, maxc_ref,
                  bases, based):
    i = pl.program_id(0)

    @pl.when(i == 0)
    def _init():
        bases[...] = jnp.zeros((1, N), _f32)
        based[...] = jnp.zeros((1, N), _f32)

    ion = lax.broadcasted_iota(_i32, (BE, N), 1)
    ii = lax.broadcasted_iota(_i32, (BE, BE), 0)
    jj = lax.broadcasted_iota(_i32, (BE, BE), 1)
    ltm = (jj < ii).astype(_f32)

    def one(col_ref, row_ref, base_ref, rnk_ref, cnt_ref):
        s_col = col_ref[...]
        s_row = row_ref[...].reshape(1, BE)
        oh = (s_col == ion).astype(_f32)
        base = base_ref[...]
        rbase = jnp.sum(oh * base, axis=1, keepdims=True)
        rloc = jnp.sum((s_col == s_row).astype(_f32) * ltm,
                       axis=1, keepdims=True)
        rnk_ref[...] = (rbase + rloc).astype(_i32)
        newb = base + jnp.sum(oh, axis=0, keepdims=True)
        base_ref[...] = newb
        cnt_ref[...] = newb.astype(_i32)
        return jnp.max(newb)

    ms = one(sc_ref, sr_ref, bases, rnks_ref, cnts_ref)
    md = one(dc_ref, dr_ref, based, rnkd_ref, cntd_ref)
    maxc_ref[...] = jnp.full((1, 128), 1.0) * jnp.maximum(ms, md)


def _tc_ranks(src2, srcr, dst2, dstr):
    return pl.pallas_call(
        _ranks_kernel,
        grid=(NBLK,),
        in_specs=[
            pl.BlockSpec((BE, 1), lambda i: (i, 0)),
            pl.BlockSpec((1, 1, BE), lambda i: (i, 0, 0)),
            pl.BlockSpec((BE, 1), lambda i: (i, 0)),
            pl.BlockSpec((1, 1, BE), lambda i: (i, 0, 0)),
        ],
        out_specs=[
            pl.BlockSpec((BE, 1), lambda i: (i, 0)),
            pl.BlockSpec((BE, 1), lambda i: (i, 0)),
            pl.BlockSpec((1, N), lambda i: (0, 0)),
            pl.BlockSpec((1, N), lambda i: (0, 0)),
            pl.BlockSpec((1, 128), lambda i: (0, 0)),
        ],
        out_shape=[
            jax.ShapeDtypeStruct((E, 1), _i32),
            jax.ShapeDtypeStruct((E, 1), _i32),
            jax.ShapeDtypeStruct((1, N), _i32),
            jax.ShapeDtypeStruct((1, N), _i32),
            jax.ShapeDtypeStruct((1, 128), _f32),
        ],
        scratch_shapes=[
            pltpu.VMEM((1, N), _f32),
            pltpu.VMEM((1, N), _f32),
        ],
    )(src2, srcr, dst2, dstr)


def _lstm_kernel(spos_s, spos_d, cnts, cntd, wih, whh, bih, bhh, scal,
                 hs_i, cs_i, hd_i, cd_i,
                 hs, cs, hd, cd, xs, xd, sems, semd):
    lo = scal[0]
    kmax = scal[1]

    def step(k, carry):
        cp1 = pltpu.make_async_copy(spos_s.at[pl.ds(k * N, N), :], xs, sems)
        cp2 = pltpu.make_async_copy(spos_d.at[pl.ds(k * N, N), :], xd, semd)
        cp1.start()
        cp2.start()
        cp1.wait()
        cp2.wait()
        p = lo + k
        for (ho, co, xb, cn_ref) in ((hs, cs, xs, cnts), (hd, cd, xd, cntd)):
            h = ho[...]
            c = co[...]
            x = xb[...]
            gates = (lax.dot_general(x, wih[...], (((1,), (1,)), ((), ())),
                                     preferred_element_type=_f32)
                     + lax.dot_general(h, whh[...], (((1,), (1,)), ((), ())),
                                       preferred_element_type=_f32)
                     + bih[...] + bhh[...])
            ig = jax.nn.sigmoid(gates[:, 0:128])
            fg = jax.nn.sigmoid(gates[:, 128:256])
            gg = jnp.tanh(gates[:, 256:384])
            og = jax.nn.sigmoid(gates[:, 384:512])
            c_new = fg * c + ig * gg
            h_new = og * jnp.tanh(c_new)
            act = cn_ref[...] > p
            ho[...] = jnp.where(act, h_new, h)
            co[...] = jnp.where(act, c_new, c)
        return carry

    lax.fori_loop(0, kmax, step, 0)


def _tc_lstm(spos_s, spos_d, cnts, cntd, W_ih, W_hh, bih2, bhh2, scal,
             hs, cs, hd, cd):
    return pl.pallas_call(
        _lstm_kernel,
        in_specs=[
            pl.BlockSpec(memory_space=pl.ANY),
            pl.BlockSpec(memory_space=pl.ANY),
            pl.BlockSpec((N, 1), lambda: (0, 0)),
            pl.BlockSpec((N, 1), lambda: (0, 0)),
            pl.BlockSpec((512, 128), lambda: (0, 0)),
            pl.BlockSpec((512, 128), lambda: (0, 0)),
            pl.BlockSpec((1, 512), lambda: (0, 0)),
            pl.BlockSpec((1, 512), lambda: (0, 0)),
            pl.BlockSpec(memory_space=pltpu.SMEM),
            pl.BlockSpec((N, D), lambda: (0, 0)),
            pl.BlockSpec((N, D), lambda: (0, 0)),
            pl.BlockSpec((N, D), lambda: (0, 0)),
            pl.BlockSpec((N, D), lambda: (0, 0)),
        ],
        out_specs=[pl.BlockSpec((N, D), lambda: (0, 0))] * 4,
        out_shape=[jax.ShapeDtypeStruct((N, D), _f32)] * 4,
        scratch_shapes=[
            pltpu.VMEM((N, D), _f32),
            pltpu.VMEM((N, D), _f32),
            pltpu.SemaphoreType.DMA,
            pltpu.SemaphoreType.DMA,
        ],
        input_output_aliases={9: 0, 10: 1, 11: 2, 12: 3},
    )(spos_s, spos_d, cnts, cntd, W_ih, W_hh, bih2, bhh2, scal,
      hs, cs, hd, cd)


def _head_kernel(x_ref, hs_ref, hd_ref, cs_ref, cd_ref,
                 w1x, w1e, b1, g1, be1, wo, bo, out_ref):
    emb = jnp.where(cd_ref[...] > 0, hd_ref[...],
                    jnp.where(cs_ref[...] > 0, hs_ref[...], 0.0))
    h1 = (lax.dot_general(x_ref[...], w1x[...], (((1,), (1,)), ((), ())),
                          preferred_element_type=_f32)
          + lax.dot_general(emb, w1e[...], (((1,), (1,)), ((), ())),
                            preferred_element_type=_f32)
          + b1[...])
    mu = jnp.mean(h1, axis=1, keepdims=True)
    var = jnp.mean((h1 - mu) ** 2, axis=1, keepdims=True)
    h1 = (h1 - mu) * lax.rsqrt(var + 1e-5) * g1[...] + be1[...]
    h1 = jnp.maximum(h1, 0.0)
    out_ref[...] = (lax.dot_general(h1, wo[...], (((1,), (1,)), ((), ())),
                                    preferred_element_type=_f32) + bo[...])


def _tc_head(x, hs, hd, cnts, cntd, w1x, w1e, b1, g1, be1, wo_p, bo_p):
    return pl.pallas_call(
        _head_kernel,
        in_specs=[
            pl.BlockSpec((N, 128), lambda: (0, 0)),
            pl.BlockSpec((N, D), lambda: (0, 0)),
            pl.BlockSpec((N, D), lambda: (0, 0)),
            pl.BlockSpec((N, 1), lambda: (0, 0)),
            pl.BlockSpec((N, 1), lambda: (0, 0)),
            pl.BlockSpec((128, 128), lambda: (0, 0)),
            pl.BlockSpec((128, 128), lambda: (0, 0)),
            pl.BlockSpec((1, 128), lambda: (0, 0)),
            pl.BlockSpec((1, 128), lambda: (0, 0)),
            pl.BlockSpec((1, 128), lambda: (0, 0)),
            pl.BlockSpec((8, 128), lambda: (0, 0)),
            pl.BlockSpec((1, 8), lambda: (0, 0)),
        ],
        out_specs=pl.BlockSpec((N, 8), lambda: (0, 0)),
        out_shape=jax.ShapeDtypeStruct((N, 8), _f32),
    )(x, hs, hd, cnts, cntd, w1x, w1e, b1, g1, be1, wo_p, bo_p)


def kernel(x, edge_src, edge_dst, heads, rels, tails, weeks, days,
           ent_embs, rel_embs, w_freq, w_phi, w_amp, d_freq, d_phi, d_amp,
           W_ih, W_hh, b_ih, b_hh, fc1_W, fc1_b, ln1_g, ln1_b, out_W, out_b):
    def padw(t):
        return jnp.pad(t, ((0, 0), (0, 128 - t.shape[1])))

    big = jnp.concatenate(
        [padw(ent_embs), padw(w_freq), padw(w_phi), padw(w_amp),
         padw(d_freq), padw(d_phi), padw(d_amp)], axis=0)

    heads = heads.astype(_i32)
    tails = tails.astype(_i32)
    rels = rels.astype(_i32)
    edge_src = edge_src.astype(_i32)
    edge_dst = edge_dst.astype(_i32)

    gath = _sc_gather(big, heads, tails)
    scores = _tc_scores(gath, weeks[:, None], days[:, None],
                        rels[:, None], rel_embs)

    src2 = edge_src[:, None]
    dst2 = edge_dst[:, None]
    srcr = edge_src.reshape(NBLK, 1, BE)
    dstr = edge_dst.reshape(NBLK, 1, BE)
    rank_s, rank_d, cnts, cntd, maxc = _tc_ranks(src2, srcr, dst2, dstr)
    maxc_sc = maxc[0, 0].astype(_i32)
    rank_s1 = rank_s.reshape(E)
    rank_d1 = rank_d.reshape(E)
    cnts_c = cnts.reshape(N, 1)
    cntd_c = cntd.reshape(N, 1)

    bih2 = b_ih[None, :]
    bhh2 = b_hh[None, :]
    z = jnp.zeros((N, D), _f32)

    def cond(st):
        lo, _hs, _cs, _hd, _cd = st
        return lo < maxc_sc

    def body(st):
        lo, hs, cs, hd, cd = st
        lo_vec = jnp.full((16,), lo, _i32)
        spos_s, spos_d = _sc_scatter(scores, rank_s1, edge_src,
                                     rank_d1, edge_dst, lo_vec)
        kmax = jnp.minimum(jnp.int32(KWIN), maxc_sc - lo)
        scal = jnp.stack([lo, kmax])
        hs, cs, hd, cd = _tc_lstm(spos_s, spos_d, cnts_c, cntd_c,
                                  W_ih, W_hh, bih2, bhh2, scal,
                                  hs, cs, hd, cd)
        return lo + KWIN, hs, cs, hd, cd

    _lo, hs, _cs, hd, _cd = lax.while_loop(
        cond, body, (jnp.int32(0), z, z, z, z))

    w1x = fc1_W[:, :128]
    w1e = fc1_W[:, 128:]
    wo_p = jnp.pad(out_W, ((0, 6), (0, 0)))
    bo_p = jnp.pad(out_b, (0, 6))[None, :]
    out = _tc_head(x, hs, hd, cnts_c, cntd_c, w1x, w1e,
                   fc1_b[None, :], ln1_g[None, :], ln1_b[None, :],
                   wo_p, bo_p)
    return out[:, :2]
